# baseline (speedup 1.0000x reference)
import functools

import jax
import jax.numpy as jnp
from jax import lax
from jax.experimental import pallas as pl
from jax.experimental.pallas import tpu as pltpu
from jax.experimental.pallas import tpu_sc as plsc

B, C, N, K = 8, 3, 4096, 20
OC = 64
R = 512
R2 = 4096
RM = 512
LANES16 = 16
NW = 32
ROWS = B * K * N // 128
RPW = ROWS // NW
NBUF = 4

_F32 = jnp.float32
_HI = lax.Precision.HIGHEST


def _dot(x, y, dims):
    return lax.dot_general(x, y, (dims, ((), ())),
                           precision=_HI, preferred_element_type=_F32)


def _lrelu(x):
    return jnp.where(x >= 0, x, 0.2 * x)


CH = 512
NCH = N // CH


def _knn(a_ref, b_ref, idx_ref, d_ref, g_ref):
    at = a_ref[0]
    ones3 = jnp.ones((3, 1), _F32)
    qn = _dot(ones3, at * at, ((0,), (0,)))
    iota_c = lax.broadcasted_iota(jnp.int32, (CH, R), 0)

    def build(c, carry):
        btc = b_ref[0, :, pl.ds(c * CH, CH)]
        cross = lax.dot_general(btc, at, ((((0,), (0,)), ((), ()))),
                                precision=lax.Precision.DEFAULT,
                                preferred_element_type=_F32)
        bn = _dot(btc * btc, ones3, ((0,), (0,)))
        d_ref[pl.ds(c * CH, CH), :] = bn + qn - 2.0 * cross
        g_ref[pl.ds(c * CH, CH), :] = iota_c + c * CH
        return carry

    lax.fori_loop(0, NCH, build, 0)

    minf = jnp.full((1, R), jnp.inf, _F32)
    jinit = jnp.full((1, R), -1, jnp.int32)
    rows = [jinit] * K

    def extract(k, carry):
        jprev, rows = carry

        def find(c, st):
            m, j = st
            gidx = g_ref[pl.ds(c * CH, CH), :]
            ch = d_ref[pl.ds(c * CH, CH), :]
            ch = jnp.where(gidx == jprev, jnp.inf, ch)
            d_ref[pl.ds(c * CH, CH), :] = ch
            mc = jnp.min(ch, axis=0, keepdims=True)
            jc = jnp.min(jnp.where(ch == mc, gidx, N), axis=0,
                         keepdims=True)
            better = mc < m
            return jnp.where(better, mc, m), jnp.where(better, jc, j)

        _, j = lax.fori_loop(0, NCH, find, (minf, jinit))
        rows = tuple(jnp.where(k == i, j, rows[i]) for i in range(K))
        return j, rows

    _, rows = lax.fori_loop(0, K, extract, (jinit, tuple(rows)))
    b_id = pl.program_id(0)
    idx_ref[0] = jnp.concatenate(list(rows), axis=0) + b_id * N


def _run_knn(a, b):
    return pl.pallas_call(
        _knn,
        grid=(B, N // R),
        in_specs=[
            pl.BlockSpec((1, C, R), lambda bi, ni: (bi, 0, ni)),
            pl.BlockSpec((1, C, N), lambda bi, ni: (bi, 0, 0)),
        ],
        out_specs=pl.BlockSpec((1, K, R), lambda bi, ni: (bi, 0, ni)),
        out_shape=jax.ShapeDtypeStruct((B, K, N), jnp.int32),
        scratch_shapes=[pltpu.VMEM((N, R), _F32),
                        pltpu.VMEM((N, R), jnp.int32)],
    )(a, b)


def _sc_gather_body(tab_hbm, idx_hbm, out_hbm, idx_v, rows_v, s0, s1, s2, s3):
    sems = (s0, s1, s2, s3)
    wid = lax.axis_index("s") * 2 + lax.axis_index("c")
    base = wid * RPW
    pltpu.sync_copy(idx_hbm.at[pl.ds(base, RPW)], idx_v)
    for p in range(NBUF):
        pltpu.async_copy(tab_hbm.at[idx_v.at[p]], rows_v.at[p], sems[p])

    def outer(i, carry):
        for p in range(NBUF):
            j = i * NBUF + p
            pltpu.make_async_copy(tab_hbm.at[pl.ds(0, 128)],
                                  rows_v.at[p], sems[p]).wait()
            pltpu.sync_copy(rows_v.at[p], out_hbm.at[base + j])

            @pl.when(j + NBUF < RPW)
            def _issue():
                pltpu.async_copy(tab_hbm.at[idx_v.at[j + NBUF]],
                                 rows_v.at[p], sems[p])
        return carry

    lax.fori_loop(0, RPW // NBUF, outer, 0)


def _gather_rows(tab, idx2d):
    fn = pl.kernel(
        _sc_gather_body,
        out_type=jax.ShapeDtypeStruct((ROWS, 128, LANES16), _F32),
        mesh=plsc.VectorSubcoreMesh(core_axis_name="c", subcore_axis_name="s"),
        scratch_types=[
            pltpu.VMEM((RPW, 128), jnp.int32),
            pltpu.VMEM((NBUF, 128, LANES16), _F32),
            pltpu.SemaphoreType.DMA,
            pltpu.SemaphoreType.DMA,
            pltpu.SemaphoreType.DMA,
            pltpu.SemaphoreType.DMA,
        ],
        compiler_params=pltpu.CompilerParams(use_tc_tiling_on_sc=False),
    )
    return fn(tab, idx2d)


def _mom1(x_ref, a_ref, hvv_ref, hqn_ref, hqq_ref, xp_ref):
    @pl.when((pl.program_id(0) == 0) & (pl.program_id(1) == 0))
    def _init():
        hvv_ref[...] = jnp.zeros_like(hvv_ref)
        hqn_ref[...] = jnp.zeros_like(hqn_ref)
        hqq_ref[...] = jnp.zeros_like(hqq_ref)

    nbr = x_ref[0]
    for k in range(K):
        xp_ref[0, k] = jnp.transpose(nbr[k], (1, 0))[0:3, :]
    e3 = (lax.broadcasted_iota(jnp.int32, (1, 1, LANES16), 2) == 3
          ).astype(_F32)
    v3 = nbr + e3
    v2 = v3.reshape(K * RM, LANES16)
    q8 = jnp.concatenate([a_ref[0], jnp.zeros((5, RM), _F32)], axis=0)
    vk = jnp.sum(v3, axis=0)
    hvv_ref[...] += _dot(v2, v2, ((0,), (0,)))
    hqn_ref[...] += _dot(q8, vk, ((1,), (0,)))
    hqq_ref[...] += float(K) * _dot(q8, q8, ((1,), (1,)))


def _run_mom1(x, a):
    return pl.pallas_call(
        _mom1,
        grid=(B, N // RM),
        in_specs=[
            pl.BlockSpec((1, K, RM, LANES16), lambda bi, ni: (bi, 0, ni, 0)),
            pl.BlockSpec((1, C, RM), lambda bi, ni: (bi, 0, ni)),
        ],
        out_specs=[
            pl.BlockSpec((LANES16, LANES16), lambda bi, ni: (0, 0)),
            pl.BlockSpec((8, LANES16), lambda bi, ni: (0, 0)),
            pl.BlockSpec((8, 8), lambda bi, ni: (0, 0)),
            pl.BlockSpec((1, K, C, RM), lambda bi, ni: (bi, 0, 0, ni)),
        ],
        out_shape=[
            jax.ShapeDtypeStruct((LANES16, LANES16), _F32),
            jax.ShapeDtypeStruct((8, LANES16), _F32),
            jax.ShapeDtypeStruct((8, 8), _F32),
            jax.ShapeDtypeStruct((B, K, C, N), _F32),
        ],
    )(x, a)


def _layer1(xp_ref, k, qy, wd_ref):
    nbrk = xp_ref[0, k]
    zt = _dot(wd_ref[...], nbrk, ((0,), (0,)))
    return _lrelu(zt + qy)


def _mom2(xp_ref, a_ref, wd_ref, wq_ref, b1_ref, s2_ref, sy_ref):
    @pl.when((pl.program_id(0) == 0) & (pl.program_id(1) == 0))
    def _init():
        s2_ref[...] = jnp.zeros_like(s2_ref)
        sy_ref[...] = jnp.zeros_like(sy_ref)

    q8 = jnp.concatenate([a_ref[0], jnp.zeros((5, R2), _F32)], axis=0)
    qy = _dot(wq_ref[...], q8, ((0,), (0,))) + b1_ref[...]
    ones8 = jnp.ones((R2, 8), _F32)

    def body(k, carry):
        s2, sy = carry
        yt = _layer1(xp_ref, k, qy, wd_ref)
        return (s2 + _dot(yt, yt, ((1,), (1,))),
                sy + _dot(yt, ones8, ((1,), (0,))))

    s2, sy = lax.fori_loop(
        0, K, body, (jnp.zeros((OC, OC), _F32), jnp.zeros((OC, 8), _F32)))
    s2_ref[...] += s2
    sy_ref[...] += sy


def _run_mom2(xp, a, wd, wq, b1c):
    return pl.pallas_call(
        _mom2,
        grid=(B, N // R2),
        in_specs=[
            pl.BlockSpec((1, K, C, R2), lambda bi, ni: (bi, 0, 0, ni)),
            pl.BlockSpec((1, C, R2), lambda bi, ni: (bi, 0, ni)),
            pl.BlockSpec((C, OC), lambda bi, ni: (0, 0)),
            pl.BlockSpec((8, OC), lambda bi, ni: (0, 0)),
            pl.BlockSpec((OC, 1), lambda bi, ni: (0, 0)),
        ],
        out_specs=[
            pl.BlockSpec((OC, OC), lambda bi, ni: (0, 0)),
            pl.BlockSpec((OC, 8), lambda bi, ni: (0, 0)),
        ],
        out_shape=[
            jax.ShapeDtypeStruct((OC, OC), _F32),
            jax.ShapeDtypeStruct((OC, 8), _F32),
        ],
    )(xp, a, wd, wq, b1c)


def _final(xp_ref, a_ref, wd_ref, wq_ref, b1_ref, w2_ref, b2_ref, o_ref):
    q8 = jnp.concatenate([a_ref[0], jnp.zeros((5, R2), _F32)], axis=0)
    qy = _dot(wq_ref[...], q8, ((0,), (0,))) + b1_ref[...]

    def body(k, acc):
        yt = _layer1(xp_ref, k, qy, wd_ref)
        zt = _lrelu(_dot(w2_ref[...], yt, ((1,), (0,))) + b2_ref[...])
        return jnp.maximum(acc, zt)

    o_ref[0] = lax.fori_loop(
        0, K, body, jnp.full((OC, R2), -jnp.inf, _F32))


def _run_final(xp, a, wd, wq, b1c, w2f, b2c):
    return pl.pallas_call(
        _final,
        grid=(B, N // R2),
        in_specs=[
            pl.BlockSpec((1, K, C, R2), lambda bi, ni: (bi, 0, 0, ni)),
            pl.BlockSpec((1, C, R2), lambda bi, ni: (bi, 0, ni)),
            pl.BlockSpec((C, OC), lambda bi, ni: (0, 0)),
            pl.BlockSpec((8, OC), lambda bi, ni: (0, 0)),
            pl.BlockSpec((OC, 1), lambda bi, ni: (0, 0)),
            pl.BlockSpec((OC, OC), lambda bi, ni: (0, 0)),
            pl.BlockSpec((OC, 1), lambda bi, ni: (0, 0)),
        ],
        out_specs=pl.BlockSpec((1, OC, R2), lambda bi, ni: (bi, 0, ni)),
        out_shape=jax.ShapeDtypeStruct((B, OC, N), _F32),
    )(xp, a, wd, wq, b1c, w2f, b2c)


def kernel(a, b, W1, gamma1, beta1, W2, gamma2, beta2):
    cnt = float(B * N * K)

    idx = _run_knn(a, b)

    ap_flat = jnp.transpose(a, (0, 2, 1)).reshape(B * N, C)
    tab = jnp.concatenate(
        [ap_flat, jnp.zeros((B * N, LANES16 - C), _F32)], axis=1)
    xr = _gather_rows(tab, idx.reshape(ROWS, 128))
    x = xr.reshape(B, K, N, LANES16)

    hvv, hqn, hqq, xp = _run_mom1(x, a)
    snn, sn = hvv[0:3, 0:3], hvv[0:3, 3]
    sqn, sq = hqn[0:3, 0:3], hqn[0:3, 3]
    sqq = hqq[0:3, 0:3]
    h7 = jnp.block([
        [snn, sqn.T, sn[:, None]],
        [sqn, sqq, sq[:, None]],
        [sn[None, :], sq[None, :], jnp.full((1, 1), cnt, _F32)],
    ])
    i3, z3 = jnp.eye(3, dtype=_F32), jnp.zeros((3, 3), _F32)
    z31, o11 = jnp.zeros((3, 1), _F32), jnp.ones((1, 1), _F32)
    m7 = jnp.block([[i3, -i3, z31], [z3, i3, z31],
                    [jnp.zeros((1, 6), _F32), o11]])
    g = m7 @ h7 @ m7.T
    mean1 = g[0:6, 6] / cnt
    cov1 = g[0:6, 0:6] / cnt - jnp.outer(mean1, mean1)
    mu1 = W1 @ mean1
    var1 = jnp.einsum('oc,cd,od->o', W1, cov1, W1)
    scale1 = gamma1 / jnp.sqrt(var1 + 1e-5)
    w1f = W1 * scale1[:, None]
    b1c = (beta1 - mu1 * scale1)[:, None]
    wd = w1f[:, 0:3].T
    wq = jnp.zeros((8, OC), _F32).at[0:3, :].set((w1f[:, 3:6] - w1f[:, 0:3]).T)

    s2, sy = _run_mom2(xp, a, wd, wq, b1c)
    mean_y = sy[:, 0] / cnt
    eyy = s2 / cnt
    mu2 = W2 @ mean_y
    var2 = jnp.einsum('oc,cd,od->o', W2, eyy, W2) - mu2 * mu2
    scale2 = gamma2 / jnp.sqrt(var2 + 1e-5)
    w2f = W2 * scale2[:, None]
    b2c = (beta2 - mu2 * scale2)[:, None]

    return _run_final(xp, a, wd, wq, b1c, w2f, b2c)


# speedup vs baseline: 1.1109x; 1.1109x over previous
import functools

import jax
import jax.numpy as jnp
from jax import lax
from jax.experimental import pallas as pl
from jax.experimental.pallas import tpu as pltpu
from jax.experimental.pallas import tpu_sc as plsc

B, C, N, K = 8, 3, 4096, 20
OC = 64
R = 1024
R2 = 4096
RM = 512
LANES16 = 16
NW = 32
ROWS = B * K * N // 128
RPW = ROWS // NW
NBUF = 4

_F32 = jnp.float32
_HI = lax.Precision.HIGHEST


def _dot(x, y, dims):
    return lax.dot_general(x, y, (dims, ((), ())),
                           precision=_HI, preferred_element_type=_F32)


def _lrelu(x):
    return jnp.where(x >= 0, x, 0.2 * x)


CH = 512
NCH = N // CH


def _knn(a_ref, b_ref, idx_ref, d_ref):
    at = a_ref[0]
    ones3 = jnp.ones((3, 1), _F32)
    qn = _dot(ones3, at * at, ((0,), (0,)))
    iota_c = lax.broadcasted_iota(jnp.int32, (CH, R), 0)

    def build(c, carry):
        btc = b_ref[0, :, pl.ds(c * CH, CH)]
        cross = lax.dot_general(btc, at, ((((0,), (0,)), ((), ()))),
                                precision=lax.Precision.DEFAULT,
                                preferred_element_type=_F32)
        bn = _dot(btc * btc, ones3, ((0,), (0,)))
        d_ref[pl.ds(c * CH, CH), :] = bn + qn - 2.0 * cross
        return carry

    lax.fori_loop(0, NCH, build, 0)

    minf = jnp.full((1, R), jnp.inf, _F32)
    jinit = jnp.full((1, R), -1, jnp.int32)
    rows = [jinit] * K

    def extract(k, carry):
        jprev, rows = carry

        def find(c, st):
            m, j = st
            gidx = iota_c + c * CH
            ch = d_ref[pl.ds(c * CH, CH), :]
            ch = jnp.where(gidx == jprev, jnp.inf, ch)
            d_ref[pl.ds(c * CH, CH), :] = ch
            mc = jnp.min(ch, axis=0, keepdims=True)
            jc = jnp.min(jnp.where(ch == mc, gidx, N), axis=0,
                         keepdims=True)
            better = mc < m
            return jnp.where(better, mc, m), jnp.where(better, jc, j)

        _, j = lax.fori_loop(0, NCH, find, (minf, jinit))
        rows = tuple(jnp.where(k == i, j, rows[i]) for i in range(K))
        return j, rows

    _, rows = lax.fori_loop(0, K, extract, (jinit, tuple(rows)))
    b_id = pl.program_id(0)
    idx_ref[0] = jnp.concatenate(list(rows), axis=0) + b_id * N


def _run_knn(a, b):
    return pl.pallas_call(
        _knn,
        grid=(B, N // R),
        in_specs=[
            pl.BlockSpec((1, C, R), lambda bi, ni: (bi, 0, ni)),
            pl.BlockSpec((1, C, N), lambda bi, ni: (bi, 0, 0)),
        ],
        out_specs=pl.BlockSpec((1, K, R), lambda bi, ni: (bi, 0, ni)),
        out_shape=jax.ShapeDtypeStruct((B, K, N), jnp.int32),
        scratch_shapes=[pltpu.VMEM((N, R), _F32)],
    )(a, b)


def _sc_gather_body(tab_hbm, idx_hbm, out_hbm, idx_v, rows_v, s0, s1, s2, s3):
    sems = (s0, s1, s2, s3)
    wid = lax.axis_index("s") * 2 + lax.axis_index("c")
    base = wid * RPW
    pltpu.sync_copy(idx_hbm.at[pl.ds(base, RPW)], idx_v)
    for p in range(NBUF):
        pltpu.async_copy(tab_hbm.at[idx_v.at[p]], rows_v.at[p], sems[p])

    def outer(i, carry):
        for p in range(NBUF):
            j = i * NBUF + p
            pltpu.make_async_copy(tab_hbm.at[pl.ds(0, 128)],
                                  rows_v.at[p], sems[p]).wait()
            pltpu.sync_copy(rows_v.at[p], out_hbm.at[base + j])

            @pl.when(j + NBUF < RPW)
            def _issue():
                pltpu.async_copy(tab_hbm.at[idx_v.at[j + NBUF]],
                                 rows_v.at[p], sems[p])
        return carry

    lax.fori_loop(0, RPW // NBUF, outer, 0)


def _gather_rows(tab, idx2d):
    fn = pl.kernel(
        _sc_gather_body,
        out_type=jax.ShapeDtypeStruct((ROWS, 128, LANES16), _F32),
        mesh=plsc.VectorSubcoreMesh(core_axis_name="c", subcore_axis_name="s"),
        scratch_types=[
            pltpu.VMEM((RPW, 128), jnp.int32),
            pltpu.VMEM((NBUF, 128, LANES16), _F32),
            pltpu.SemaphoreType.DMA,
            pltpu.SemaphoreType.DMA,
            pltpu.SemaphoreType.DMA,
            pltpu.SemaphoreType.DMA,
        ],
        compiler_params=pltpu.CompilerParams(use_tc_tiling_on_sc=False),
    )
    return fn(tab, idx2d)


def _mom1(x_ref, a_ref, hvv_ref, hqn_ref, hqq_ref, xp_ref):
    @pl.when((pl.program_id(0) == 0) & (pl.program_id(1) == 0))
    def _init():
        hvv_ref[...] = jnp.zeros_like(hvv_ref)
        hqn_ref[...] = jnp.zeros_like(hqn_ref)
        hqq_ref[...] = jnp.zeros_like(hqq_ref)

    nbr = x_ref[0]
    for k in range(K):
        xp_ref[0, k] = jnp.transpose(nbr[k], (1, 0))[0:3, :]
    e3 = (lax.broadcasted_iota(jnp.int32, (1, 1, LANES16), 2) == 3
          ).astype(_F32)
    v3 = nbr + e3
    v2 = v3.reshape(K * RM, LANES16)
    q8 = jnp.concatenate([a_ref[0], jnp.zeros((5, RM), _F32)], axis=0)
    vk = jnp.sum(v3, axis=0)
    hvv_ref[...] += _dot(v2, v2, ((0,), (0,)))
    hqn_ref[...] += _dot(q8, vk, ((1,), (0,)))
    hqq_ref[...] += float(K) * _dot(q8, q8, ((1,), (1,)))


def _run_mom1(x, a):
    return pl.pallas_call(
        _mom1,
        grid=(B, N // RM),
        in_specs=[
            pl.BlockSpec((1, K, RM, LANES16), lambda bi, ni: (bi, 0, ni, 0)),
            pl.BlockSpec((1, C, RM), lambda bi, ni: (bi, 0, ni)),
        ],
        out_specs=[
            pl.BlockSpec((LANES16, LANES16), lambda bi, ni: (0, 0)),
            pl.BlockSpec((8, LANES16), lambda bi, ni: (0, 0)),
            pl.BlockSpec((8, 8), lambda bi, ni: (0, 0)),
            pl.BlockSpec((1, K, C, RM), lambda bi, ni: (bi, 0, 0, ni)),
        ],
        out_shape=[
            jax.ShapeDtypeStruct((LANES16, LANES16), _F32),
            jax.ShapeDtypeStruct((8, LANES16), _F32),
            jax.ShapeDtypeStruct((8, 8), _F32),
            jax.ShapeDtypeStruct((B, K, C, N), _F32),
        ],
    )(x, a)


def _layer1(xp_ref, k, qy, wd_ref):
    nbrk = xp_ref[0, k]
    zt = _dot(wd_ref[...], nbrk, ((0,), (0,)))
    return _lrelu(zt + qy)


def _mom2(xp_ref, a_ref, wd_ref, wq_ref, b1_ref, s2_ref, sy_ref):
    @pl.when((pl.program_id(0) == 0) & (pl.program_id(1) == 0))
    def _init():
        s2_ref[...] = jnp.zeros_like(s2_ref)
        sy_ref[...] = jnp.zeros_like(sy_ref)

    q8 = jnp.concatenate([a_ref[0], jnp.zeros((5, R2), _F32)], axis=0)
    qy = _dot(wq_ref[...], q8, ((0,), (0,))) + b1_ref[...]
    ones8 = jnp.ones((R2, 8), _F32)

    def body(k, carry):
        s2, sy = carry
        yt = _layer1(xp_ref, k, qy, wd_ref)
        return (s2 + _dot(yt, yt, ((1,), (1,))),
                sy + _dot(yt, ones8, ((1,), (0,))))

    s2, sy = lax.fori_loop(
        0, K, body, (jnp.zeros((OC, OC), _F32), jnp.zeros((OC, 8), _F32)))
    s2_ref[...] += s2
    sy_ref[...] += sy


def _run_mom2(xp, a, wd, wq, b1c):
    return pl.pallas_call(
        _mom2,
        grid=(B, N // R2),
        in_specs=[
            pl.BlockSpec((1, K, C, R2), lambda bi, ni: (bi, 0, 0, ni)),
            pl.BlockSpec((1, C, R2), lambda bi, ni: (bi, 0, ni)),
            pl.BlockSpec((C, OC), lambda bi, ni: (0, 0)),
            pl.BlockSpec((8, OC), lambda bi, ni: (0, 0)),
            pl.BlockSpec((OC, 1), lambda bi, ni: (0, 0)),
        ],
        out_specs=[
            pl.BlockSpec((OC, OC), lambda bi, ni: (0, 0)),
            pl.BlockSpec((OC, 8), lambda bi, ni: (0, 0)),
        ],
        out_shape=[
            jax.ShapeDtypeStruct((OC, OC), _F32),
            jax.ShapeDtypeStruct((OC, 8), _F32),
        ],
    )(xp, a, wd, wq, b1c)


def _final(xp_ref, a_ref, wd_ref, wq_ref, b1_ref, w2_ref, b2_ref, o_ref):
    q8 = jnp.concatenate([a_ref[0], jnp.zeros((5, R2), _F32)], axis=0)
    qy = _dot(wq_ref[...], q8, ((0,), (0,))) + b1_ref[...]

    def body(k, acc):
        yt = _layer1(xp_ref, k, qy, wd_ref)
        zt = _lrelu(_dot(w2_ref[...], yt, ((1,), (0,))) + b2_ref[...])
        return jnp.maximum(acc, zt)

    o_ref[0] = lax.fori_loop(
        0, K, body, jnp.full((OC, R2), -jnp.inf, _F32))


def _run_final(xp, a, wd, wq, b1c, w2f, b2c):
    return pl.pallas_call(
        _final,
        grid=(B, N // R2),
        in_specs=[
            pl.BlockSpec((1, K, C, R2), lambda bi, ni: (bi, 0, 0, ni)),
            pl.BlockSpec((1, C, R2), lambda bi, ni: (bi, 0, ni)),
            pl.BlockSpec((C, OC), lambda bi, ni: (0, 0)),
            pl.BlockSpec((8, OC), lambda bi, ni: (0, 0)),
            pl.BlockSpec((OC, 1), lambda bi, ni: (0, 0)),
            pl.BlockSpec((OC, OC), lambda bi, ni: (0, 0)),
            pl.BlockSpec((OC, 1), lambda bi, ni: (0, 0)),
        ],
        out_specs=pl.BlockSpec((1, OC, R2), lambda bi, ni: (bi, 0, ni)),
        out_shape=jax.ShapeDtypeStruct((B, OC, N), _F32),
    )(xp, a, wd, wq, b1c, w2f, b2c)


def kernel(a, b, W1, gamma1, beta1, W2, gamma2, beta2):
    cnt = float(B * N * K)

    idx = _run_knn(a, b)

    ap_flat = jnp.transpose(a, (0, 2, 1)).reshape(B * N, C)
    tab = jnp.concatenate(
        [ap_flat, jnp.zeros((B * N, LANES16 - C), _F32)], axis=1)
    xr = _gather_rows(tab, idx.reshape(ROWS, 128))
    x = xr.reshape(B, K, N, LANES16)

    hvv, hqn, hqq, xp = _run_mom1(x, a)
    snn, sn = hvv[0:3, 0:3], hvv[0:3, 3]
    sqn, sq = hqn[0:3, 0:3], hqn[0:3, 3]
    sqq = hqq[0:3, 0:3]
    h7 = jnp.block([
        [snn, sqn.T, sn[:, None]],
        [sqn, sqq, sq[:, None]],
        [sn[None, :], sq[None, :], jnp.full((1, 1), cnt, _F32)],
    ])
    i3, z3 = jnp.eye(3, dtype=_F32), jnp.zeros((3, 3), _F32)
    z31, o11 = jnp.zeros((3, 1), _F32), jnp.ones((1, 1), _F32)
    m7 = jnp.block([[i3, -i3, z31], [z3, i3, z31],
                    [jnp.zeros((1, 6), _F32), o11]])
    g = m7 @ h7 @ m7.T
    mean1 = g[0:6, 6] / cnt
    cov1 = g[0:6, 0:6] / cnt - jnp.outer(mean1, mean1)
    mu1 = W1 @ mean1
    var1 = jnp.einsum('oc,cd,od->o', W1, cov1, W1)
    scale1 = gamma1 / jnp.sqrt(var1 + 1e-5)
    w1f = W1 * scale1[:, None]
    b1c = (beta1 - mu1 * scale1)[:, None]
    wd = w1f[:, 0:3].T
    wq = jnp.zeros((8, OC), _F32).at[0:3, :].set((w1f[:, 3:6] - w1f[:, 0:3]).T)

    s2, sy = _run_mom2(xp, a, wd, wq, b1c)
    mean_y = sy[:, 0] / cnt
    eyy = s2 / cnt
    mu2 = W2 @ mean_y
    var2 = jnp.einsum('oc,cd,od->o', W2, eyy, W2) - mu2 * mu2
    scale2 = gamma2 / jnp.sqrt(var2 + 1e-5)
    w2f = W2 * scale2[:, None]
    b2c = (beta2 - mu2 * scale2)[:, None]

    return _run_final(xp, a, wd, wq, b1c, w2f, b2c)


# speedup vs baseline: 1.1424x; 1.0283x over previous
import functools

import jax
import jax.numpy as jnp
from jax import lax
from jax.experimental import pallas as pl
from jax.experimental.pallas import tpu as pltpu
from jax.experimental.pallas import tpu_sc as plsc

B, C, N, K = 8, 3, 4096, 20
OC = 64
R = 2048
R2 = 4096
RM = 512
LANES16 = 16
NW = 32
ROWS = B * K * N // 128
RPW = ROWS // NW
NBUF = 4

_F32 = jnp.float32
_HI = lax.Precision.HIGHEST


def _dot(x, y, dims):
    return lax.dot_general(x, y, (dims, ((), ())),
                           precision=_HI, preferred_element_type=_F32)


def _lrelu(x):
    return jnp.where(x >= 0, x, 0.2 * x)


CH = 512
NCH = N // CH


def _knn(a_ref, b_ref, idx_ref, d_ref):
    at = a_ref[0]
    ones3 = jnp.ones((3, 1), _F32)
    qn = _dot(ones3, at * at, ((0,), (0,)))
    iota_c = lax.broadcasted_iota(jnp.int32, (CH, R), 0)

    def build(c, carry):
        btc = b_ref[0, :, pl.ds(c * CH, CH)]
        cross = lax.dot_general(btc, at, ((((0,), (0,)), ((), ()))),
                                precision=lax.Precision.DEFAULT,
                                preferred_element_type=_F32)
        bn = _dot(btc * btc, ones3, ((0,), (0,)))
        d_ref[pl.ds(c * CH, CH), :] = bn + qn - 2.0 * cross
        return carry

    lax.fori_loop(0, NCH, build, 0)

    minf = jnp.full((1, R), jnp.inf, _F32)
    jinit = jnp.full((1, R), -1, jnp.int32)
    rows = [jinit] * K

    def extract(k, carry):
        jprev, rows = carry

        def find(c, st):
            m, j = st
            gidx = iota_c + c * CH
            ch = d_ref[pl.ds(c * CH, CH), :]
            ch = jnp.where(gidx == jprev, jnp.inf, ch)
            d_ref[pl.ds(c * CH, CH), :] = ch
            mc = jnp.min(ch, axis=0, keepdims=True)
            jc = jnp.min(jnp.where(ch == mc, gidx, N), axis=0,
                         keepdims=True)
            better = mc < m
            return jnp.where(better, mc, m), jnp.where(better, jc, j)

        _, j = lax.fori_loop(0, NCH, find, (minf, jinit))
        rows = tuple(jnp.where(k == i, j, rows[i]) for i in range(K))
        return j, rows

    _, rows = lax.fori_loop(0, K, extract, (jinit, tuple(rows)))
    b_id = pl.program_id(0)
    idx_ref[0] = jnp.concatenate(list(rows), axis=0) + b_id * N


def _run_knn(a, b):
    return pl.pallas_call(
        _knn,
        grid=(B, N // R),
        in_specs=[
            pl.BlockSpec((1, C, R), lambda bi, ni: (bi, 0, ni)),
            pl.BlockSpec((1, C, N), lambda bi, ni: (bi, 0, 0)),
        ],
        out_specs=pl.BlockSpec((1, K, R), lambda bi, ni: (bi, 0, ni)),
        out_shape=jax.ShapeDtypeStruct((B, K, N), jnp.int32),
        scratch_shapes=[pltpu.VMEM((N, R), _F32)],
    )(a, b)


def _sc_gather_body(tab_hbm, idx_hbm, out_hbm, idx_v, rows_v, s0, s1, s2, s3):
    sems = (s0, s1, s2, s3)
    wid = lax.axis_index("s") * 2 + lax.axis_index("c")
    base = wid * RPW
    pltpu.sync_copy(idx_hbm.at[pl.ds(base, RPW)], idx_v)
    for p in range(NBUF):
        pltpu.async_copy(tab_hbm.at[idx_v.at[p]], rows_v.at[p], sems[p])

    def outer(i, carry):
        for p in range(NBUF):
            j = i * NBUF + p
            pltpu.make_async_copy(tab_hbm.at[pl.ds(0, 128)],
                                  rows_v.at[p], sems[p]).wait()
            pltpu.sync_copy(rows_v.at[p], out_hbm.at[base + j])

            @pl.when(j + NBUF < RPW)
            def _issue():
                pltpu.async_copy(tab_hbm.at[idx_v.at[j + NBUF]],
                                 rows_v.at[p], sems[p])
        return carry

    lax.fori_loop(0, RPW // NBUF, outer, 0)


def _gather_rows(tab, idx2d):
    fn = pl.kernel(
        _sc_gather_body,
        out_type=jax.ShapeDtypeStruct((ROWS, 128, LANES16), _F32),
        mesh=plsc.VectorSubcoreMesh(core_axis_name="c", subcore_axis_name="s"),
        scratch_types=[
            pltpu.VMEM((RPW, 128), jnp.int32),
            pltpu.VMEM((NBUF, 128, LANES16), _F32),
            pltpu.SemaphoreType.DMA,
            pltpu.SemaphoreType.DMA,
            pltpu.SemaphoreType.DMA,
            pltpu.SemaphoreType.DMA,
        ],
        compiler_params=pltpu.CompilerParams(use_tc_tiling_on_sc=False),
    )
    return fn(tab, idx2d)


def _mom1(x_ref, a_ref, hvv_ref, hqn_ref, hqq_ref, xp_ref):
    @pl.when((pl.program_id(0) == 0) & (pl.program_id(1) == 0))
    def _init():
        hvv_ref[...] = jnp.zeros_like(hvv_ref)
        hqn_ref[...] = jnp.zeros_like(hqn_ref)
        hqq_ref[...] = jnp.zeros_like(hqq_ref)

    nbr = x_ref[0]
    for k in range(K):
        xp_ref[0, k] = jnp.transpose(nbr[k], (1, 0))[0:3, :]
    e3 = (lax.broadcasted_iota(jnp.int32, (1, 1, LANES16), 2) == 3
          ).astype(_F32)
    v3 = nbr + e3
    v2 = v3.reshape(K * RM, LANES16)
    q8 = jnp.concatenate([a_ref[0], jnp.zeros((5, RM), _F32)], axis=0)
    vk = jnp.sum(v3, axis=0)
    hvv_ref[...] += _dot(v2, v2, ((0,), (0,)))
    hqn_ref[...] += _dot(q8, vk, ((1,), (0,)))
    hqq_ref[...] += float(K) * _dot(q8, q8, ((1,), (1,)))


def _run_mom1(x, a):
    return pl.pallas_call(
        _mom1,
        grid=(B, N // RM),
        in_specs=[
            pl.BlockSpec((1, K, RM, LANES16), lambda bi, ni: (bi, 0, ni, 0)),
            pl.BlockSpec((1, C, RM), lambda bi, ni: (bi, 0, ni)),
        ],
        out_specs=[
            pl.BlockSpec((LANES16, LANES16), lambda bi, ni: (0, 0)),
            pl.BlockSpec((8, LANES16), lambda bi, ni: (0, 0)),
            pl.BlockSpec((8, 8), lambda bi, ni: (0, 0)),
            pl.BlockSpec((1, K, C, RM), lambda bi, ni: (bi, 0, 0, ni)),
        ],
        out_shape=[
            jax.ShapeDtypeStruct((LANES16, LANES16), _F32),
            jax.ShapeDtypeStruct((8, LANES16), _F32),
            jax.ShapeDtypeStruct((8, 8), _F32),
            jax.ShapeDtypeStruct((B, K, C, N), _F32),
        ],
    )(x, a)


def _layer1(xp_ref, k, qy, wd_ref):
    nbrk = xp_ref[0, k]
    zt = _dot(wd_ref[...], nbrk, ((0,), (0,)))
    return _lrelu(zt + qy)


def _mom2(xp_ref, a_ref, wd_ref, wq_ref, b1_ref, s2_ref, sy_ref):
    @pl.when((pl.program_id(0) == 0) & (pl.program_id(1) == 0))
    def _init():
        s2_ref[...] = jnp.zeros_like(s2_ref)
        sy_ref[...] = jnp.zeros_like(sy_ref)

    q8 = jnp.concatenate([a_ref[0], jnp.zeros((5, R2), _F32)], axis=0)
    qy = _dot(wq_ref[...], q8, ((0,), (0,))) + b1_ref[...]
    ones8 = jnp.ones((R2, 8), _F32)

    def body(k, carry):
        s2, sy = carry
        yt = _layer1(xp_ref, k, qy, wd_ref)
        return (s2 + _dot(yt, yt, ((1,), (1,))),
                sy + _dot(yt, ones8, ((1,), (0,))))

    s2, sy = lax.fori_loop(
        0, K, body, (jnp.zeros((OC, OC), _F32), jnp.zeros((OC, 8), _F32)))
    s2_ref[...] += s2
    sy_ref[...] += sy


def _run_mom2(xp, a, wd, wq, b1c):
    return pl.pallas_call(
        _mom2,
        grid=(B, N // R2),
        in_specs=[
            pl.BlockSpec((1, K, C, R2), lambda bi, ni: (bi, 0, 0, ni)),
            pl.BlockSpec((1, C, R2), lambda bi, ni: (bi, 0, ni)),
            pl.BlockSpec((C, OC), lambda bi, ni: (0, 0)),
            pl.BlockSpec((8, OC), lambda bi, ni: (0, 0)),
            pl.BlockSpec((OC, 1), lambda bi, ni: (0, 0)),
        ],
        out_specs=[
            pl.BlockSpec((OC, OC), lambda bi, ni: (0, 0)),
            pl.BlockSpec((OC, 8), lambda bi, ni: (0, 0)),
        ],
        out_shape=[
            jax.ShapeDtypeStruct((OC, OC), _F32),
            jax.ShapeDtypeStruct((OC, 8), _F32),
        ],
    )(xp, a, wd, wq, b1c)


def _final(xp_ref, a_ref, wd_ref, wq_ref, b1_ref, w2_ref, b2_ref, o_ref):
    q8 = jnp.concatenate([a_ref[0], jnp.zeros((5, R2), _F32)], axis=0)
    qy = _dot(wq_ref[...], q8, ((0,), (0,))) + b1_ref[...]

    def body(k, acc):
        yt = _layer1(xp_ref, k, qy, wd_ref)
        zt = _lrelu(_dot(w2_ref[...], yt, ((1,), (0,))) + b2_ref[...])
        return jnp.maximum(acc, zt)

    o_ref[0] = lax.fori_loop(
        0, K, body, jnp.full((OC, R2), -jnp.inf, _F32))


def _run_final(xp, a, wd, wq, b1c, w2f, b2c):
    return pl.pallas_call(
        _final,
        grid=(B, N // R2),
        in_specs=[
            pl.BlockSpec((1, K, C, R2), lambda bi, ni: (bi, 0, 0, ni)),
            pl.BlockSpec((1, C, R2), lambda bi, ni: (bi, 0, ni)),
            pl.BlockSpec((C, OC), lambda bi, ni: (0, 0)),
            pl.BlockSpec((8, OC), lambda bi, ni: (0, 0)),
            pl.BlockSpec((OC, 1), lambda bi, ni: (0, 0)),
            pl.BlockSpec((OC, OC), lambda bi, ni: (0, 0)),
            pl.BlockSpec((OC, 1), lambda bi, ni: (0, 0)),
        ],
        out_specs=pl.BlockSpec((1, OC, R2), lambda bi, ni: (bi, 0, ni)),
        out_shape=jax.ShapeDtypeStruct((B, OC, N), _F32),
    )(xp, a, wd, wq, b1c, w2f, b2c)


def kernel(a, b, W1, gamma1, beta1, W2, gamma2, beta2):
    cnt = float(B * N * K)

    idx = _run_knn(a, b)

    ap_flat = jnp.transpose(a, (0, 2, 1)).reshape(B * N, C)
    tab = jnp.concatenate(
        [ap_flat, jnp.zeros((B * N, LANES16 - C), _F32)], axis=1)
    xr = _gather_rows(tab, idx.reshape(ROWS, 128))
    x = xr.reshape(B, K, N, LANES16)

    hvv, hqn, hqq, xp = _run_mom1(x, a)
    snn, sn = hvv[0:3, 0:3], hvv[0:3, 3]
    sqn, sq = hqn[0:3, 0:3], hqn[0:3, 3]
    sqq = hqq[0:3, 0:3]
    h7 = jnp.block([
        [snn, sqn.T, sn[:, None]],
        [sqn, sqq, sq[:, None]],
        [sn[None, :], sq[None, :], jnp.full((1, 1), cnt, _F32)],
    ])
    i3, z3 = jnp.eye(3, dtype=_F32), jnp.zeros((3, 3), _F32)
    z31, o11 = jnp.zeros((3, 1), _F32), jnp.ones((1, 1), _F32)
    m7 = jnp.block([[i3, -i3, z31], [z3, i3, z31],
                    [jnp.zeros((1, 6), _F32), o11]])
    g = m7 @ h7 @ m7.T
    mean1 = g[0:6, 6] / cnt
    cov1 = g[0:6, 0:6] / cnt - jnp.outer(mean1, mean1)
    mu1 = W1 @ mean1
    var1 = jnp.einsum('oc,cd,od->o', W1, cov1, W1)
    scale1 = gamma1 / jnp.sqrt(var1 + 1e-5)
    w1f = W1 * scale1[:, None]
    b1c = (beta1 - mu1 * scale1)[:, None]
    wd = w1f[:, 0:3].T
    wq = jnp.zeros((8, OC), _F32).at[0:3, :].set((w1f[:, 3:6] - w1f[:, 0:3]).T)

    s2, sy = _run_mom2(xp, a, wd, wq, b1c)
    mean_y = sy[:, 0] / cnt
    eyy = s2 / cnt
    mu2 = W2 @ mean_y
    var2 = jnp.einsum('oc,cd,od->o', W2, eyy, W2) - mu2 * mu2
    scale2 = gamma2 / jnp.sqrt(var2 + 1e-5)
    w2f = W2 * scale2[:, None]
    b2c = (beta2 - mu2 * scale2)[:, None]

    return _run_final(xp, a, wd, wq, b1c, w2f, b2c)
